# Initial kernel scaffold; baseline (speedup 1.0000x reference)
import jax
import jax.numpy as jnp
from jax.experimental import pallas as pl


def kernel(x, edge_index, edge_attr, batch, edge_enc_W, edge_enc_b, pre_W, pre_b, post_W, post_b, lin_W, lin_b, bn_gamma, bn_beta, mlp_W1, mlp_b1, mlp_W2, mlp_b2, mlp_W3, mlp_b3):
    raise NotImplementedError("write your pallas kernel here")


# speedup vs baseline: 9.9594x; 9.9594x over previous
import jax
import jax.numpy as jnp
import numpy as _np
from jax.experimental import pallas as pl
from jax.experimental.pallas import tpu as pltpu

N = 10000
E = 160000
F = 128
T = 8
F_OUT = 4
HID = 32
N_GRAPHS = 64
EPS = 1e-5
TF = T * F

_DEG_HIST = _np.array([0.0] * 8 + [625.0] * 16 + [0.0] * 9)
_bins = _np.arange(_DEG_HIST.shape[0], dtype=_np.float64)
AVG_LOG = float((_np.log(_bins + 1.0) * _DEG_HIST).sum() / _DEG_HIST.sum())

NB = 400


def _cs_body(x_ref, w1_ref, w2_ref, c0_ref, c_ref, s_ref):
    xb = x_ref[...]
    c_ref[...] = jnp.dot(xb, w1_ref[...], preferred_element_type=jnp.float32) + c0_ref[...]
    s_ref[...] = jnp.dot(xb, w2_ref[...], preferred_element_type=jnp.float32)


def _compute_c_s(x, w1r, w2r, c0):
    return pl.pallas_call(
        _cs_body,
        grid=(N // NB,),
        in_specs=[
            pl.BlockSpec((NB, F), lambda i: (i, 0)),
            pl.BlockSpec((F, TF), lambda i: (0, 0)),
            pl.BlockSpec((F, TF), lambda i: (0, 0)),
            pl.BlockSpec((1, TF), lambda i: (0, 0)),
        ],
        out_specs=[
            pl.BlockSpec((NB, TF), lambda i: (i, 0)),
            pl.BlockSpec((NB, TF), lambda i: (i, 0)),
        ],
        out_shape=[
            jax.ShapeDtypeStruct((N, TF), jnp.float32),
            jax.ShapeDtypeStruct((N, TF), jnp.float32),
        ],
    )(x, w1r, w2r, c0)


def _post_body(x_ref, c_ref, q1_ref, q2_ref, qmin_ref, qmax_ref, deg_ref,
               wx_ref, pw_ref, pb_ref, lw_ref, lb_ref, o_ref, s1_ref, s2_ref):
    deg = deg_ref[...]
    degc = jnp.maximum(deg, 1.0)
    C = c_ref[...]
    Q1 = q1_ref[...]
    mean = (deg * C + Q1) / degc
    e2 = (deg * C * C + 2.0 * C * Q1 + q2_ref[...]) / degc
    std = jnp.sqrt(jax.nn.relu(e2 - mean * mean) + EPS)
    has = deg > 0
    mn = jnp.where(has, C + qmin_ref[...], 0.0)
    mx = jnp.where(has, C + qmax_ref[...], 0.0)
    amp = jnp.log(deg + 1.0) / AVG_LOG
    att = AVG_LOG / jnp.log(degc + 1.0)
    acc = jnp.dot(x_ref[...], wx_ref[...], preferred_element_type=jnp.float32)
    parts = (mean, mn, mx, std)
    for k in range(4):
        a = parts[k]
        yp = jnp.dot(a, pw_ref[k * TF:(k + 1) * TF, :],
                     preferred_element_type=jnp.float32)
        ya = jnp.dot(a, pw_ref[(4 + k) * TF:(5 + k) * TF, :],
                     preferred_element_type=jnp.float32)
        yt = jnp.dot(a, pw_ref[(8 + k) * TF:(9 + k) * TF, :],
                     preferred_element_type=jnp.float32)
        acc += yp + amp * ya + att * yt
    acc += pb_ref[...]
    out = jnp.dot(acc, lw_ref[...], preferred_element_type=jnp.float32) + lb_ref[...]
    o_ref[...] = out

    @pl.when(pl.program_id(0) == 0)
    def _init():
        s1_ref[...] = jnp.zeros_like(s1_ref)
        s2_ref[...] = jnp.zeros_like(s2_ref)

    s1_ref[...] += jnp.sum(out, axis=0, keepdims=True)
    s2_ref[...] += jnp.sum(out * out, axis=0, keepdims=True)


def _post(x, C, Q1, Q2, Qmin, Qmax, deg2d, wx, pw_packed, pb, lin_W, lin_b):
    return pl.pallas_call(
        _post_body,
        grid=(N // NB,),
        in_specs=[
            pl.BlockSpec((NB, F), lambda i: (i, 0)),
            pl.BlockSpec((NB, TF), lambda i: (i, 0)),
            pl.BlockSpec((NB, TF), lambda i: (i, 0)),
            pl.BlockSpec((NB, TF), lambda i: (i, 0)),
            pl.BlockSpec((NB, TF), lambda i: (i, 0)),
            pl.BlockSpec((NB, TF), lambda i: (i, 0)),
            pl.BlockSpec((NB, 1), lambda i: (i, 0)),
            pl.BlockSpec((F, HID), lambda i: (0, 0)),
            pl.BlockSpec((12 * TF, HID), lambda i: (0, 0)),
            pl.BlockSpec((1, HID), lambda i: (0, 0)),
            pl.BlockSpec((HID, HID), lambda i: (0, 0)),
            pl.BlockSpec((1, HID), lambda i: (0, 0)),
        ],
        out_specs=[
            pl.BlockSpec((NB, HID), lambda i: (i, 0)),
            pl.BlockSpec((1, HID), lambda i: (0, 0)),
            pl.BlockSpec((1, HID), lambda i: (0, 0)),
        ],
        out_shape=[
            jax.ShapeDtypeStruct((N, HID), jnp.float32),
            jax.ShapeDtypeStruct((1, HID), jnp.float32),
            jax.ShapeDtypeStruct((1, HID), jnp.float32),
        ],
    )(x, C, Q1, Q2, Qmin, Qmax, deg2d, wx, pw_packed, pb, lin_W, lin_b)


PAD = 8


def _final_body(o_ref, s1_ref, s2_ref, oneh_ref, bg_ref, bb_ref,
                w1_ref, b1_ref, w2_ref, b2_ref, w3_ref, b3_ref, out_ref,
                acc_ref):
    i = pl.program_id(0)
    mu = s1_ref[...] / N
    var = s2_ref[...] / N - mu * mu
    o = (o_ref[...] - mu) / jnp.sqrt(var + EPS) * bg_ref[...] + bb_ref[...]
    o = jax.nn.relu(o)

    @pl.when(i == 0)
    def _init():
        acc_ref[...] = jnp.zeros_like(acc_ref)

    acc_ref[...] += jax.lax.dot_general(
        oneh_ref[...], o, (((0,), (0,)), ((), ())),
        preferred_element_type=jnp.float32)

    @pl.when(i == pl.num_programs(0) - 1)
    def _fin():
        g = acc_ref[...]
        g = jax.nn.relu(jnp.dot(g, w1_ref[...], preferred_element_type=jnp.float32) + b1_ref[...])
        g = jax.nn.relu(jnp.dot(g, w2_ref[...], preferred_element_type=jnp.float32) + b2_ref[...])
        g = jnp.dot(g, w3_ref[...], preferred_element_type=jnp.float32) + b3_ref[...]
        col = jax.lax.broadcasted_iota(jnp.int32, (N_GRAPHS, PAD), 1)
        g = jnp.where(col < 2, g, -1e30)
        m = jnp.max(g, axis=-1, keepdims=True)
        s = jnp.log(jnp.sum(jnp.exp(g - m), axis=-1, keepdims=True))
        out_ref[...] = g - m - s


def _final(o, s1, s2, oneh, bn_gamma, bn_beta, mlp_W1, mlp_b1, mlp_W2, mlp_b2,
           w3p, b3p):
    return pl.pallas_call(
        _final_body,
        grid=(N // NB,),
        in_specs=[
            pl.BlockSpec((NB, HID), lambda i: (i, 0)),
            pl.BlockSpec((1, HID), lambda i: (0, 0)),
            pl.BlockSpec((1, HID), lambda i: (0, 0)),
            pl.BlockSpec((NB, N_GRAPHS), lambda i: (i, 0)),
            pl.BlockSpec((1, HID), lambda i: (0, 0)),
            pl.BlockSpec((1, HID), lambda i: (0, 0)),
            pl.BlockSpec((HID, HID // 2), lambda i: (0, 0)),
            pl.BlockSpec((1, HID // 2), lambda i: (0, 0)),
            pl.BlockSpec((HID // 2, HID // 4), lambda i: (0, 0)),
            pl.BlockSpec((1, HID // 4), lambda i: (0, 0)),
            pl.BlockSpec((HID // 4, PAD), lambda i: (0, 0)),
            pl.BlockSpec((1, PAD), lambda i: (0, 0)),
        ],
        out_specs=pl.BlockSpec((N_GRAPHS, PAD), lambda i: (0, 0)),
        out_shape=jax.ShapeDtypeStruct((N_GRAPHS, PAD), jnp.float32),
        scratch_shapes=[pltpu.VMEM((N_GRAPHS, HID), jnp.float32)],
    )(o, s1, s2, oneh, bn_gamma, bn_beta, mlp_W1, mlp_b1, mlp_W2, mlp_b2,
      w3p, b3p)


def kernel(x, edge_index, edge_attr, batch, edge_enc_W, edge_enc_b, pre_W,
           pre_b, post_W, post_b, lin_W, lin_b, bn_gamma, bn_beta, mlp_W1,
           mlp_b1, mlp_W2, mlp_b2, mlp_W3, mlp_b3):
    src, dst = edge_index[0], edge_index[1]
    W1 = pre_W[:, :F, :]
    W2 = pre_W[:, F:2 * F, :]
    W3 = pre_W[:, 2 * F:, :]
    v = jnp.einsum('f,tfo->to', edge_enc_W[0], W3).reshape(-1)
    c0 = (jnp.einsum('f,tfo->to', edge_enc_b, W3) + pre_b).reshape(1, -1)
    W1r = W1.transpose(1, 0, 2).reshape(F, TF)
    W2r = W2.transpose(1, 0, 2).reshape(F, TF)
    C, S = _compute_c_s(x, W1r, W2r, c0)

    q = S[src] + edge_attr[:, None] * v[None]
    deg = jax.ops.segment_sum(jnp.ones((E,), jnp.float32), dst, N)
    Q1 = jax.ops.segment_sum(q, dst, N)
    Q2 = jax.ops.segment_sum(q * q, dst, N)
    Qmin = jax.ops.segment_min(q, dst, N)
    Qmax = jax.ops.segment_max(q, dst, N)
    hasv = (deg > 0)[:, None]
    Qmin = jnp.where(hasv, Qmin, 0.0)
    Qmax = jnp.where(hasv, Qmax, 0.0)

    wx = post_W[:, :F, :].transpose(1, 0, 2).reshape(F, T * F_OUT)
    eye_t = jnp.eye(T, dtype=jnp.float32)
    def bd(p):
        w = post_W[:, p * F:(p + 1) * F, :]
        wb = jnp.einsum('tfo,ts->tfso', w, eye_t)
        return wb.reshape(TF, T * F_OUT)
    order = [1, 2, 3, 4, 5, 6, 7, 8, 9, 10, 11, 12]
    pw_packed = jnp.concatenate([bd(p) for p in order], axis=0)
    pb = post_b.reshape(1, HID)

    o, s1, s2 = _post(x, C, Q1, Q2, Qmin, Qmax,
                      deg.reshape(N, 1), wx, pw_packed, pb, lin_W,
                      lin_b.reshape(1, -1))

    oneh = (batch[:, None] == jnp.arange(N_GRAPHS, dtype=batch.dtype)[None, :]).astype(jnp.float32)
    w3p = jnp.zeros((HID // 4, PAD), jnp.float32).at[:, :2].set(mlp_W3)
    b3p = jnp.zeros((1, PAD), jnp.float32).at[:, :2].set(mlp_b3)
    outp = _final(o, s1, s2, oneh, bn_gamma.reshape(1, HID),
                  bn_beta.reshape(1, HID), mlp_W1, mlp_b1.reshape(1, -1),
                  mlp_W2, mlp_b2.reshape(1, -1), w3p, b3p)
    return outp[:, :2]
